# baseline (speedup 1.0000x reference)
import functools

import jax
import jax.numpy as jnp
from jax import lax
from jax.experimental import pallas as pl
from jax.experimental.pallas import tpu as pltpu
from jax.experimental.pallas import tpu_sc as plsc

N = 10000
E = 160000
NW = 32
EPW = E // NW
CB = 2000
SBUF = 4096
SEGSZ = 10256
DPW = 320

_mesh = lambda: plsc.VectorSubcoreMesh(core_axis_name="c", subcore_axis_name="s")


def _wid():
    return lax.axis_index("c") * 16 + lax.axis_index("s")


NGRP = (EPW + 15) // 16


@functools.partial(
    pl.kernel,
    out_type=jax.ShapeDtypeStruct((NW * N,), jnp.int32),
    mesh=_mesh(),
    compiler_params=pltpu.CompilerParams(needs_layout_passes=False),
    scratch_types=[pltpu.VMEM((EPW + 16,), jnp.int32),
                   pltpu.VMEM((N,), jnp.int32)],
)
def _hist_kernel(dst_hbm, hist_hbm, dstbuf, cnt):
    w = _wid()
    dstbuf[pl.ds(EPW, 16)] = jnp.zeros((16,), jnp.int32)
    pltpu.sync_copy(dst_hbm.at[pl.ds(w * EPW, EPW)], dstbuf.at[pl.ds(0, EPW)])

    def zbody(i, _):
        cnt[pl.ds(i * 16, 16)] = jnp.zeros((16,), jnp.int32)
        return 0

    lax.fori_loop(0, N // 16, zbody, 0)
    lanes = lax.iota(jnp.int32, 16)

    def ebody(j, _):
        idx = dstbuf[pl.ds(j * 16, 16)]
        valid = (j * 16 + lanes) < EPW
        g = plsc.load_gather(cnt, [idx])
        rc, last = plsc.scan_count(idx, mask=valid)
        plsc.store_scatter(cnt, [idx], g + rc, mask=last)
        return 0

    lax.fori_loop(0, NGRP, ebody, 0)
    pltpu.sync_copy(cnt, hist_hbm.at[pl.ds(w * N, N)])


@functools.partial(
    pl.kernel,
    out_type=(
        jax.ShapeDtypeStruct((NW * N,), jnp.int32),
        jax.ShapeDtypeStruct((SEGSZ,), jnp.int32),
    ),
    mesh=_mesh(),
    compiler_params=pltpu.CompilerParams(needs_layout_passes=False),
    scratch_types=[
        pltpu.VMEM((NW * CB,), jnp.int32),
        pltpu.VMEM((CB,), jnp.int32),
        pltpu.VMEM((CB,), jnp.int32),
        pltpu.VMEM((16,), jnp.int32),
    ],
)
def _scan_kernel(hist_hbm, base_hbm, seg_hbm, bpart, rowbuf, colsum, tmp16):
    w = _wid()

    @pl.when(w == 0)
    def _():
        def chunk_body(ch, running):
            def z(i, _):
                colsum[pl.ds(i * 16, 16)] = jnp.zeros((16,), jnp.int32)
                return 0

            lax.fori_loop(0, CB // 16, z, 0)
            for s in range(NW):
                pltpu.sync_copy(hist_hbm.at[pl.ds(s * N + ch * CB, CB)], rowbuf)

                def acc(i, _):
                    v = colsum[pl.ds(i * 16, 16)]
                    bpart[pl.ds(s * CB + i * 16, 16)] = v
                    colsum[pl.ds(i * 16, 16)] = v + rowbuf[pl.ds(i * 16, 16)]
                    return 0

                lax.fori_loop(0, CB // 16, acc, 0)

            def sbody(i, run):
                v = colsum[pl.ds(i * 16, 16)]
                cs = plsc.cumsum(v)
                colsum[pl.ds(i * 16, 16)] = cs - v + run
                return run + jnp.sum(v)

            run2 = lax.fori_loop(0, CB // 16, sbody, running)
            pltpu.sync_copy(colsum, seg_hbm.at[pl.ds(ch * CB, CB)])
            for s in range(NW):
                def fin(i, _):
                    bpart[pl.ds(s * CB + i * 16, 16)] = (
                        bpart[pl.ds(s * CB + i * 16, 16)]
                        + colsum[pl.ds(i * 16, 16)]
                    )
                    return 0

                lax.fori_loop(0, CB // 16, fin, 0)
                pltpu.sync_copy(bpart.at[pl.ds(s * CB, CB)],
                                base_hbm.at[pl.ds(s * N + ch * CB, CB)])
            return run2

        lax.fori_loop(0, N // CB, chunk_body, 0)
        tmp16[...] = jnp.full((16,), E, jnp.int32)
        pltpu.sync_copy(tmp16, seg_hbm.at[pl.ds(N, 16)])


@functools.partial(
    pl.kernel,
    out_type=(
        jax.ShapeDtypeStruct((E,), jnp.int32),
        jax.ShapeDtypeStruct((E,), jnp.int32),
    ),
    mesh=_mesh(),
    compiler_params=pltpu.CompilerParams(needs_layout_passes=False),
    scratch_types=[
        pltpu.VMEM((EPW + 16,), jnp.int32),
        pltpu.VMEM((EPW + 16,), jnp.int32),
        pltpu.VMEM((N,), jnp.int32),
        pltpu.VMEM((EPW + 16,), jnp.int32),
    ],
)
def _place_kernel(dst_hbm, src_hbm, base_hbm, srcs_hbm, dsts_hbm,
                  dstbuf, srcbuf, basebuf, posbuf):
    w = _wid()
    dstbuf[pl.ds(EPW, 16)] = jnp.zeros((16,), jnp.int32)
    pltpu.sync_copy(dst_hbm.at[pl.ds(w * EPW, EPW)], dstbuf.at[pl.ds(0, EPW)])
    pltpu.sync_copy(src_hbm.at[pl.ds(w * EPW, EPW)], srcbuf.at[pl.ds(0, EPW)])
    pltpu.sync_copy(base_hbm.at[pl.ds(w * N, N)], basebuf)
    lanes = lax.iota(jnp.int32, 16)

    def ebody(j, _):
        idx = dstbuf[pl.ds(j * 16, 16)]
        valid = (j * 16 + lanes) < EPW
        g = plsc.load_gather(basebuf, [idx])
        rc, last = plsc.scan_count(idx, mask=valid)
        pos = jnp.where(valid, g + rc - 1, -1)
        posbuf[pl.ds(j * 16, 16)] = pos
        plsc.store_scatter(basebuf, [idx], g + rc, mask=last)
        return 0

    lax.fori_loop(0, NGRP, ebody, 0)

    def sbody(j, _):
        idx = posbuf[pl.ds(j * 16, 16)]
        pltpu.sync_copy(srcbuf.at[pl.ds(j * 16, 16)],
                        srcs_hbm.at[plsc.Indices(idx, ignored_value=-1)])
        pltpu.sync_copy(dstbuf.at[pl.ds(j * 16, 16)],
                        dsts_hbm.at[plsc.Indices(idx, ignored_value=-1)])
        return 0

    lax.fori_loop(0, NGRP, sbody, 0)


def _sort_edges(src, dst):
    hist = _hist_kernel(dst)
    base, seg = _scan_kernel(hist)
    srcs_s, dsts_s = _place_kernel(dst, src, base)
    return srcs_s, dsts_s, seg


def _tc_linear(h, W_src, b_src, W_dst, b_dst, leak):
    n, fin = h.shape
    C = W_src.shape[1]
    BM = 1000

    def body(h_ref, ws_ref, bs_ref, wd_ref, bd_ref, xl_ref, xr_ref):
        a = h_ref[...]
        if leak:
            a = jnp.where(a > 0, a, 0.01 * a)
        xl_ref[...] = (
            jnp.dot(a, ws_ref[...], preferred_element_type=jnp.float32)
            + bs_ref[...]
        )
        xr_ref[...] = (
            jnp.dot(a, wd_ref[...], preferred_element_type=jnp.float32)
            + bd_ref[...]
        )

    return pl.pallas_call(
        body,
        grid=(n // BM,),
        in_specs=[
            pl.BlockSpec((BM, fin), lambda i: (i, 0)),
            pl.BlockSpec((fin, C), lambda i: (0, 0)),
            pl.BlockSpec((1, C), lambda i: (0, 0)),
            pl.BlockSpec((fin, C), lambda i: (0, 0)),
            pl.BlockSpec((1, C), lambda i: (0, 0)),
        ],
        out_specs=[
            pl.BlockSpec((BM, C), lambda i: (i, 0)),
            pl.BlockSpec((BM, C), lambda i: (i, 0)),
        ],
        out_shape=[jax.ShapeDtypeStruct((n, C), jnp.float32)] * 2,
    )(h, W_src, b_src.reshape(1, C), W_dst, b_dst.reshape(1, C))


def _edge_jnp(xl, xr, srcs_s, dsts_s, seg, att, bias):
    segN = seg[: N + 1]
    dst_ids = jnp.searchsorted(segN[1:], jnp.arange(E), side="right")
    e = xl[srcs_s] + xr[dst_ids]
    e = jnp.where(e > 0, e, 0.2 * e)
    alpha = e @ att
    amax = jax.ops.segment_max(alpha, dst_ids, num_segments=N)
    amax = jnp.where(jnp.isfinite(amax), amax, 0.0)
    ex = jnp.exp(alpha - amax[dst_ids])
    denom = jax.ops.segment_sum(ex, dst_ids, num_segments=N)
    a = ex / (denom[dst_ids] + 1e-16)
    msg = xl[srcs_s] * a[:, None]
    out = jax.ops.segment_sum(msg, dst_ids, num_segments=N)
    return out + bias


def _pad_c(a, C):
    pad = C - a.shape[-1]
    if pad == 0:
        return a
    cfg = [(0, 0)] * (a.ndim - 1) + [(0, pad)]
    return jnp.pad(a, cfg)


def kernel(x, edge_index, W_src1, b_src1, W_dst1, b_dst1, att1, bias1,
           W_src2, b_src2, W_dst2, b_dst2, att2, bias2,
           W_src3, b_src3, W_dst3, b_dst3, att3, bias3):
    src = edge_index[0]
    dst = edge_index[1]
    srcs_s, dsts_s, seg = _sort_edges(src, dst)

    h = x
    layers = [
        (W_src1, b_src1, W_dst1, b_dst1, att1, bias1, 128, False),
        (W_src2, b_src2, W_dst2, b_dst2, att2, bias2, 512, True),
        (W_src3, b_src3, W_dst3, b_dst3, att3, bias3, 1040, True),
    ]
    for (Ws, bs, Wd, bd, att, bias, C, leak) in layers:
        Ws, bs, Wd, bd = (_pad_c(Ws, C), _pad_c(bs, C),
                          _pad_c(Wd, C), _pad_c(bd, C))
        attp = _pad_c(att, C)[0]
        biasp = _pad_c(bias, C)
        xl, xr = _tc_linear(h, Ws, bs, Wd, bd, leak)
        h = _edge_jnp(xl, xr, srcs_s, dsts_s, seg, attp, biasp)
    return h[:, :1028]


# speedup vs baseline: 3.1037x; 3.1037x over previous
import functools

import jax
import jax.numpy as jnp
from jax import lax
from jax.experimental import pallas as pl
from jax.experimental.pallas import tpu as pltpu
from jax.experimental.pallas import tpu_sc as plsc

N = 10000
N2 = 10240
E = 160000
NW = 32
EPW = E // NW
CB = 2000
SEGSZ = 10368
DPW = 320

_mesh = lambda: plsc.VectorSubcoreMesh(core_axis_name="c", subcore_axis_name="s")


def _wid():
    return lax.axis_index("c") * 16 + lax.axis_index("s")


NGRP = (EPW + 15) // 16


@functools.partial(
    pl.kernel,
    out_type=jax.ShapeDtypeStruct((NW * N,), jnp.int32),
    mesh=_mesh(),
    compiler_params=pltpu.CompilerParams(needs_layout_passes=False),
    scratch_types=[pltpu.VMEM((EPW + 16,), jnp.int32),
                   pltpu.VMEM((N,), jnp.int32)],
)
def _hist_kernel(dst_hbm, hist_hbm, dstbuf, cnt):
    w = _wid()
    dstbuf[pl.ds(EPW, 16)] = jnp.zeros((16,), jnp.int32)
    pltpu.sync_copy(dst_hbm.at[pl.ds(w * EPW, EPW)], dstbuf.at[pl.ds(0, EPW)])

    def zbody(i, _):
        cnt[pl.ds(i * 16, 16)] = jnp.zeros((16,), jnp.int32)
        return 0

    lax.fori_loop(0, N // 16, zbody, 0)
    lanes = lax.iota(jnp.int32, 16)

    def ebody(j, _):
        idx = dstbuf[pl.ds(j * 16, 16)]
        valid = (j * 16 + lanes) < EPW
        g = plsc.load_gather(cnt, [idx])
        rc, last = plsc.scan_count(idx, mask=valid)
        plsc.store_scatter(cnt, [idx], g + rc, mask=last)
        return 0

    lax.fori_loop(0, NGRP, ebody, 0)
    pltpu.sync_copy(cnt, hist_hbm.at[pl.ds(w * N, N)])


@functools.partial(
    pl.kernel,
    out_type=(
        jax.ShapeDtypeStruct((NW * N,), jnp.int32),
        jax.ShapeDtypeStruct((SEGSZ,), jnp.int32),
    ),
    mesh=_mesh(),
    compiler_params=pltpu.CompilerParams(needs_layout_passes=False),
    scratch_types=[
        pltpu.VMEM((NW * CB,), jnp.int32),
        pltpu.VMEM((CB,), jnp.int32),
        pltpu.VMEM((CB,), jnp.int32),
        pltpu.VMEM((16,), jnp.int32),
    ],
)
def _scan_kernel(hist_hbm, base_hbm, seg_hbm, bpart, rowbuf, colsum, tmp16):
    w = _wid()

    @pl.when(w == 0)
    def _():
        def chunk_body(ch, running):
            def z(i, _):
                colsum[pl.ds(i * 16, 16)] = jnp.zeros((16,), jnp.int32)
                return 0

            lax.fori_loop(0, CB // 16, z, 0)
            for s in range(NW):
                pltpu.sync_copy(hist_hbm.at[pl.ds(s * N + ch * CB, CB)], rowbuf)

                def acc(i, _):
                    v = colsum[pl.ds(i * 16, 16)]
                    bpart[pl.ds(s * CB + i * 16, 16)] = v
                    colsum[pl.ds(i * 16, 16)] = v + rowbuf[pl.ds(i * 16, 16)]
                    return 0

                lax.fori_loop(0, CB // 16, acc, 0)

            def sbody(i, run):
                v = colsum[pl.ds(i * 16, 16)]
                cs = plsc.cumsum(v)
                colsum[pl.ds(i * 16, 16)] = cs - v + run
                return run + jnp.sum(v)

            run2 = lax.fori_loop(0, CB // 16, sbody, running)
            pltpu.sync_copy(colsum, seg_hbm.at[pl.ds(ch * CB, CB)])
            for s in range(NW):
                def fin(i, _):
                    bpart[pl.ds(s * CB + i * 16, 16)] = (
                        bpart[pl.ds(s * CB + i * 16, 16)]
                        + colsum[pl.ds(i * 16, 16)]
                    )
                    return 0

                lax.fori_loop(0, CB // 16, fin, 0)
                pltpu.sync_copy(bpart.at[pl.ds(s * CB, CB)],
                                base_hbm.at[pl.ds(s * N + ch * CB, CB)])
            return run2

        lax.fori_loop(0, N // CB, chunk_body, 0)
        tmp16[...] = jnp.full((16,), E, jnp.int32)

        def fill(i, _):
            pltpu.sync_copy(tmp16, seg_hbm.at[pl.ds(N + i * 16, 16)])
            return 0

        lax.fori_loop(0, (SEGSZ - N) // 16, fill, 0)


@functools.partial(
    pl.kernel,
    out_type=jax.ShapeDtypeStruct((E + 16,), jnp.int32),
    mesh=_mesh(),
    compiler_params=pltpu.CompilerParams(needs_layout_passes=False),
    scratch_types=[
        pltpu.VMEM((EPW + 16,), jnp.int32),
        pltpu.VMEM((EPW + 16,), jnp.int32),
        pltpu.VMEM((N,), jnp.int32),
        pltpu.VMEM((EPW + 16,), jnp.int32),
        pltpu.VMEM((16,), jnp.int32),
    ],
)
def _place_kernel(dst_hbm, src_hbm, base_hbm, srcs_hbm,
                  dstbuf, srcbuf, basebuf, posbuf, tmp16):
    w = _wid()
    dstbuf[pl.ds(EPW, 16)] = jnp.zeros((16,), jnp.int32)
    pltpu.sync_copy(dst_hbm.at[pl.ds(w * EPW, EPW)], dstbuf.at[pl.ds(0, EPW)])
    pltpu.sync_copy(src_hbm.at[pl.ds(w * EPW, EPW)], srcbuf.at[pl.ds(0, EPW)])
    pltpu.sync_copy(base_hbm.at[pl.ds(w * N, N)], basebuf)
    lanes = lax.iota(jnp.int32, 16)

    @pl.when(w == 0)
    def _():
        tmp16[...] = jnp.zeros((16,), jnp.int32)
        pltpu.sync_copy(tmp16, srcs_hbm.at[pl.ds(E, 16)])

    def ebody(j, _):
        idx = dstbuf[pl.ds(j * 16, 16)]
        valid = (j * 16 + lanes) < EPW
        g = plsc.load_gather(basebuf, [idx])
        rc, last = plsc.scan_count(idx, mask=valid)
        pos = jnp.where(valid, g + rc - 1, -1)
        posbuf[pl.ds(j * 16, 16)] = pos
        plsc.store_scatter(basebuf, [idx], g + rc, mask=last)
        return 0

    lax.fori_loop(0, NGRP, ebody, 0)

    def sbody(j, _):
        idx = posbuf[pl.ds(j * 16, 16)]
        pltpu.sync_copy(srcbuf.at[pl.ds(j * 16, 16)],
                        srcs_hbm.at[plsc.Indices(idx, ignored_value=-1)])
        return 0

    lax.fori_loop(0, NGRP, sbody, 0)


def _sort_edges(src, dst):
    hist = _hist_kernel(dst)
    base, seg = _scan_kernel(hist)
    srcs_s = _place_kernel(dst, src, base)
    return srcs_s, seg


def _tc_linear(h, W_src, b_src, W_dst, b_dst, leak):
    n, fin = h.shape
    C = W_src.shape[1]
    BM = 1024

    def body(h_ref, ws_ref, bs_ref, wd_ref, bd_ref, xl_ref, xr_ref):
        a = h_ref[...]
        if leak:
            a = jnp.where(a > 0, a, 0.01 * a)
        xl_ref[...] = (
            jnp.dot(a, ws_ref[...], preferred_element_type=jnp.float32)
            + bs_ref[...]
        )
        xr_ref[...] = (
            jnp.dot(a, wd_ref[...], preferred_element_type=jnp.float32)
            + bd_ref[...]
        )

    return pl.pallas_call(
        body,
        grid=(n // BM,),
        in_specs=[
            pl.BlockSpec((BM, fin), lambda i: (i, 0)),
            pl.BlockSpec((fin, C), lambda i: (0, 0)),
            pl.BlockSpec((1, C), lambda i: (0, 0)),
            pl.BlockSpec((fin, C), lambda i: (0, 0)),
            pl.BlockSpec((1, C), lambda i: (0, 0)),
        ],
        out_specs=[
            pl.BlockSpec((BM, C), lambda i: (i, 0)),
            pl.BlockSpec((BM, C), lambda i: (i, 0)),
        ],
        out_shape=[jax.ShapeDtypeStruct((n, C), jnp.float32)] * 2,
    )(h, W_src, b_src.reshape(1, C), W_dst, b_dst.reshape(1, C))


@functools.cache
def _edge_kernel(C):
    nkk = C // 16

    @functools.partial(
        pl.kernel,
        out_type=jax.ShapeDtypeStruct((N2, C), jnp.float32),
        mesh=_mesh(),
        compiler_params=pltpu.CompilerParams(needs_layout_passes=False),
        scratch_types=[
            pltpu.VMEM((384,), jnp.int32),
            pltpu.VMEM((56,), jnp.int32),
            pltpu.VMEM((16,), jnp.int32),
            pltpu.VMEM((16, C), jnp.float32),
            pltpu.VMEM((16, C), jnp.float32),
            pltpu.VMEM((16, C), jnp.float32),
            pltpu.VMEM((C,), jnp.float32),
            pltpu.VMEM((C,), jnp.float32),
            pltpu.VMEM((C,), jnp.float32),
            pltpu.SemaphoreType.DMA,
        ],
    )
    def k(xl_hbm, xr_hbm, srcs_hbm, seg_hbm, att_hbm, bias_hbm, out_hbm,
          segbuf, srcwin, idxbuf, rowbuf, xrbuf, outbuf, S, attb, biasb, sem):
        w = _wid()
        d_lo = w * DPW
        pltpu.sync_copy(seg_hbm.at[pl.ds(d_lo, 384)], segbuf)
        pltpu.sync_copy(att_hbm, attb)
        pltpu.sync_copy(bias_hbm, biasb)

        def zS(i, _):
            S[pl.ds(i * 16, 16)] = jnp.zeros((16,), jnp.float32)
            return 0

        lax.fori_loop(0, nkk, zS, 0)
        minf = jnp.full((16,), -jnp.inf, jnp.float32)
        zero16 = jnp.zeros((16,), jnp.float32)

        def blk_body(blk, _):
            b0 = d_lo + blk * 16
            pltpu.sync_copy(xr_hbm.at[pl.ds(b0, 16)], xrbuf)

            def dst_body(db, _):
                rd = blk * 16 + db
                sv = segbuf[pl.ds(rd, 16)]
                e0 = sv[0]
                e1 = sv[1]
                ngr = (e1 - e0 + 15) // 16

                def group_body(g, car):
                    m, dsum = car
                    e = e0 + g * 16
                    ae = jnp.minimum((e // 8) * 8, E - 40)
                    off = e - ae
                    pltpu.sync_copy(srcs_hbm.at[pl.ds(ae, 56)], srcwin)
                    idxbuf[...] = srcwin[pl.ds(off, 16)]
                    pltpu.async_copy(xl_hbm.at[idxbuf], rowbuf, sem).wait()
                    cnt = jnp.minimum(16, e1 - e)

                    def edge_body(r, car2):
                        m2, dsum2 = car2
                        acc = zero16
                        for kk in range(nkk):
                            xlv = rowbuf[r, pl.ds(kk * 16, 16)]
                            xrv = xrbuf[db, pl.ds(kk * 16, 16)]
                            z = xlv + xrv
                            l = 0.6 * z + 0.4 * jnp.abs(z)
                            acc = acc + attb[pl.ds(kk * 16, 16)] * l
                        av = jnp.full((16,), jnp.sum(acc), jnp.float32)
                        mn = jnp.maximum(m2, av)
                        rsc = jnp.exp(m2 - mn)
                        wv = jnp.exp(av - mn)
                        dsum2 = dsum2 * rsc + wv

                        def upS(kk2, _):
                            svv = S[pl.ds(kk2 * 16, 16)]
                            xv = rowbuf[r, pl.ds(kk2 * 16, 16)]
                            S[pl.ds(kk2 * 16, 16)] = svv * rsc + wv * xv
                            return 0

                        lax.fori_loop(0, nkk, upS, 0)
                        return (mn, dsum2)

                    return lax.fori_loop(0, cnt, edge_body, (m, dsum))

                m, dsum = lax.fori_loop(0, ngr, group_body, (minf, zero16))
                rcp = jnp.where(dsum > 0, 1.0 / (dsum + 1e-16), 0.0)

                def flush(kk, _):
                    svv = S[pl.ds(kk * 16, 16)]
                    bv = biasb[pl.ds(kk * 16, 16)]
                    outbuf[db, pl.ds(kk * 16, 16)] = svv * rcp + bv
                    return 0

                lax.fori_loop(0, nkk, flush, 0)
                return 0

            lax.fori_loop(0, 16, dst_body, 0)
            pltpu.sync_copy(outbuf, out_hbm.at[pl.ds(b0, 16)])
            return 0

        lax.fori_loop(0, DPW // 16, blk_body, 0)

    return k


def _pad_c(a, C):
    pad = C - a.shape[-1]
    if pad == 0:
        return a
    cfg = [(0, 0)] * (a.ndim - 1) + [(0, pad)]
    return jnp.pad(a, cfg)


def kernel(x, edge_index, W_src1, b_src1, W_dst1, b_dst1, att1, bias1,
           W_src2, b_src2, W_dst2, b_dst2, att2, bias2,
           W_src3, b_src3, W_dst3, b_dst3, att3, bias3):
    src = edge_index[0]
    dst = edge_index[1]
    srcs_s, seg = _sort_edges(src, dst)

    h = jnp.pad(x, ((0, N2 - N), (0, 0)))
    layers = [
        (W_src1, b_src1, W_dst1, b_dst1, att1, bias1, 128, False),
        (W_src2, b_src2, W_dst2, b_dst2, att2, bias2, 512, True),
        (W_src3, b_src3, W_dst3, b_dst3, att3, bias3, 1152, True),
    ]
    for (Ws, bs, Wd, bd, att, bias, C, leak) in layers:
        Ws, bs, Wd, bd = (_pad_c(Ws, C), _pad_c(bs, C),
                          _pad_c(Wd, C), _pad_c(bd, C))
        attp = _pad_c(att, C)[0]
        biasp = _pad_c(bias, C)
        xl, xr = _tc_linear(h, Ws, bs, Wd, bd, leak)
        h = _edge_kernel(C)(xl, xr, srcs_s, seg, attp, biasp)
    return h[:N, :1028]


# speedup vs baseline: 4.9856x; 1.6063x over previous
import functools

import jax
import jax.numpy as jnp
from jax import lax
from jax.experimental import pallas as pl
from jax.experimental.pallas import tpu as pltpu
from jax.experimental.pallas import tpu_sc as plsc

N = 10000
N2 = 10240
E = 160000
NW = 32
EPW = E // NW
CB = 2000
SEGSZ = 10368
DPW = 320

_mesh = lambda: plsc.VectorSubcoreMesh(core_axis_name="c", subcore_axis_name="s")


def _wid():
    return lax.axis_index("c") * 16 + lax.axis_index("s")


NGRP = (EPW + 15) // 16


@functools.partial(
    pl.kernel,
    out_type=jax.ShapeDtypeStruct((NW * N,), jnp.int32),
    mesh=_mesh(),
    compiler_params=pltpu.CompilerParams(needs_layout_passes=False),
    scratch_types=[pltpu.VMEM((EPW + 16,), jnp.int32),
                   pltpu.VMEM((N,), jnp.int32)],
)
def _hist_kernel(dst_hbm, hist_hbm, dstbuf, cnt):
    w = _wid()
    dstbuf[pl.ds(EPW, 16)] = jnp.zeros((16,), jnp.int32)
    pltpu.sync_copy(dst_hbm.at[pl.ds(w * EPW, EPW)], dstbuf.at[pl.ds(0, EPW)])

    def zbody(i, _):
        cnt[pl.ds(i * 16, 16)] = jnp.zeros((16,), jnp.int32)
        return 0

    lax.fori_loop(0, N // 16, zbody, 0)
    lanes = lax.iota(jnp.int32, 16)

    def ebody(j, _):
        idx = dstbuf[pl.ds(j * 16, 16)]
        valid = (j * 16 + lanes) < EPW
        g = plsc.load_gather(cnt, [idx])
        rc, last = plsc.scan_count(idx, mask=valid)
        plsc.store_scatter(cnt, [idx], g + rc, mask=last)
        return 0

    lax.fori_loop(0, NGRP, ebody, 0)
    pltpu.sync_copy(cnt, hist_hbm.at[pl.ds(w * N, N)])


@functools.partial(
    pl.kernel,
    out_type=(
        jax.ShapeDtypeStruct((NW * N,), jnp.int32),
        jax.ShapeDtypeStruct((SEGSZ,), jnp.int32),
    ),
    mesh=_mesh(),
    compiler_params=pltpu.CompilerParams(needs_layout_passes=False),
    scratch_types=[
        pltpu.VMEM((NW * CB,), jnp.int32),
        pltpu.VMEM((CB,), jnp.int32),
        pltpu.VMEM((CB,), jnp.int32),
        pltpu.VMEM((16,), jnp.int32),
    ],
)
def _scan_kernel(hist_hbm, base_hbm, seg_hbm, bpart, rowbuf, colsum, tmp16):
    w = _wid()

    @pl.when(w == 0)
    def _():
        def chunk_body(ch, running):
            def z(i, _):
                colsum[pl.ds(i * 16, 16)] = jnp.zeros((16,), jnp.int32)
                return 0

            lax.fori_loop(0, CB // 16, z, 0)
            for s in range(NW):
                pltpu.sync_copy(hist_hbm.at[pl.ds(s * N + ch * CB, CB)], rowbuf)

                def acc(i, _):
                    v = colsum[pl.ds(i * 16, 16)]
                    bpart[pl.ds(s * CB + i * 16, 16)] = v
                    colsum[pl.ds(i * 16, 16)] = v + rowbuf[pl.ds(i * 16, 16)]
                    return 0

                lax.fori_loop(0, CB // 16, acc, 0)

            def sbody(i, run):
                v = colsum[pl.ds(i * 16, 16)]
                cs = plsc.cumsum(v)
                colsum[pl.ds(i * 16, 16)] = cs - v + run
                return run + jnp.sum(v)

            run2 = lax.fori_loop(0, CB // 16, sbody, running)
            pltpu.sync_copy(colsum, seg_hbm.at[pl.ds(ch * CB, CB)])
            for s in range(NW):
                def fin(i, _):
                    bpart[pl.ds(s * CB + i * 16, 16)] = (
                        bpart[pl.ds(s * CB + i * 16, 16)]
                        + colsum[pl.ds(i * 16, 16)]
                    )
                    return 0

                lax.fori_loop(0, CB // 16, fin, 0)
                pltpu.sync_copy(bpart.at[pl.ds(s * CB, CB)],
                                base_hbm.at[pl.ds(s * N + ch * CB, CB)])
            return run2

        lax.fori_loop(0, N // CB, chunk_body, 0)
        tmp16[...] = jnp.full((16,), E, jnp.int32)

        def fill(i, _):
            pltpu.sync_copy(tmp16, seg_hbm.at[pl.ds(N + i * 16, 16)])
            return 0

        lax.fori_loop(0, (SEGSZ - N) // 16, fill, 0)


@functools.partial(
    pl.kernel,
    out_type=jax.ShapeDtypeStruct((E + 16,), jnp.int32),
    mesh=_mesh(),
    compiler_params=pltpu.CompilerParams(needs_layout_passes=False),
    scratch_types=[
        pltpu.VMEM((EPW + 16,), jnp.int32),
        pltpu.VMEM((EPW + 16,), jnp.int32),
        pltpu.VMEM((N,), jnp.int32),
        pltpu.VMEM((EPW + 16,), jnp.int32),
        pltpu.VMEM((16,), jnp.int32),
    ],
)
def _place_kernel(dst_hbm, src_hbm, base_hbm, srcs_hbm,
                  dstbuf, srcbuf, basebuf, posbuf, tmp16):
    w = _wid()
    dstbuf[pl.ds(EPW, 16)] = jnp.zeros((16,), jnp.int32)
    pltpu.sync_copy(dst_hbm.at[pl.ds(w * EPW, EPW)], dstbuf.at[pl.ds(0, EPW)])
    pltpu.sync_copy(src_hbm.at[pl.ds(w * EPW, EPW)], srcbuf.at[pl.ds(0, EPW)])
    pltpu.sync_copy(base_hbm.at[pl.ds(w * N, N)], basebuf)
    lanes = lax.iota(jnp.int32, 16)

    @pl.when(w == 0)
    def _():
        tmp16[...] = jnp.zeros((16,), jnp.int32)
        pltpu.sync_copy(tmp16, srcs_hbm.at[pl.ds(E, 16)])

    def ebody(j, _):
        idx = dstbuf[pl.ds(j * 16, 16)]
        valid = (j * 16 + lanes) < EPW
        g = plsc.load_gather(basebuf, [idx])
        rc, last = plsc.scan_count(idx, mask=valid)
        pos = jnp.where(valid, g + rc - 1, -1)
        posbuf[pl.ds(j * 16, 16)] = pos
        plsc.store_scatter(basebuf, [idx], g + rc, mask=last)
        return 0

    lax.fori_loop(0, NGRP, ebody, 0)

    def sbody(j, _):
        idx = posbuf[pl.ds(j * 16, 16)]
        pltpu.sync_copy(srcbuf.at[pl.ds(j * 16, 16)],
                        srcs_hbm.at[plsc.Indices(idx, ignored_value=-1)])
        return 0

    lax.fori_loop(0, NGRP, sbody, 0)


def _sort_edges(src, dst):
    hist = _hist_kernel(dst)
    base, seg = _scan_kernel(hist)
    srcs_s = _place_kernel(dst, src, base)
    return srcs_s, seg


def _tc_linear(h, W_src, b_src, W_dst, b_dst, leak):
    n, fin = h.shape
    C = W_src.shape[1]
    BM = 1024

    def body(h_ref, ws_ref, bs_ref, wd_ref, bd_ref, xl_ref, xr_ref):
        a = h_ref[...]
        if leak:
            a = jnp.where(a > 0, a, 0.01 * a)
        xl_ref[...] = (
            jnp.dot(a, ws_ref[...], preferred_element_type=jnp.float32)
            + bs_ref[...]
        )
        xr_ref[...] = (
            jnp.dot(a, wd_ref[...], preferred_element_type=jnp.float32)
            + bd_ref[...]
        )

    return pl.pallas_call(
        body,
        grid=(n // BM,),
        in_specs=[
            pl.BlockSpec((BM, fin), lambda i: (i, 0)),
            pl.BlockSpec((fin, C), lambda i: (0, 0)),
            pl.BlockSpec((1, C), lambda i: (0, 0)),
            pl.BlockSpec((fin, C), lambda i: (0, 0)),
            pl.BlockSpec((1, C), lambda i: (0, 0)),
        ],
        out_specs=[
            pl.BlockSpec((BM, C), lambda i: (i, 0)),
            pl.BlockSpec((BM, C), lambda i: (i, 0)),
        ],
        out_shape=[jax.ShapeDtypeStruct((n, C), jnp.float32)] * 2,
    )(h, W_src, b_src.reshape(1, C), W_dst, b_dst.reshape(1, C))


SBUFSZ = 4096
WCLAMP = E + 16 - SBUFSZ


@functools.cache
def _edge_kernel(C):
    nkk = C // 16

    @functools.partial(
        pl.kernel,
        out_type=jax.ShapeDtypeStruct((N2, C), jnp.float32),
        mesh=_mesh(),
        compiler_params=pltpu.CompilerParams(needs_layout_passes=False),
        scratch_types=[
            pltpu.VMEM((384,), jnp.int32),
            pltpu.VMEM((SBUFSZ,), jnp.int32),
            pltpu.VMEM((16,), jnp.int32),
            pltpu.VMEM((16, C), jnp.float32),
            pltpu.VMEM((16, C), jnp.float32),
            pltpu.VMEM((16, C), jnp.float32),
            pltpu.VMEM((C,), jnp.float32),
            pltpu.VMEM((C,), jnp.float32),
            pltpu.VMEM((C,), jnp.float32),
            pltpu.SemaphoreType.DMA,
        ],
    )
    def k(xl_hbm, xr_hbm, srcs_hbm, seg_hbm, att_hbm, bias_hbm, out_hbm,
          segbuf, srcbuf, idxbuf, rowbuf, xrbuf, outbuf, S, attb, biasb, sem):
        w = _wid()
        d_lo = w * DPW
        pltpu.sync_copy(seg_hbm.at[pl.ds(d_lo, 384)], segbuf)
        pltpu.sync_copy(att_hbm, attb)
        pltpu.sync_copy(bias_hbm, biasb)

        def zS(i, _):
            S[pl.ds(i * 16, 16)] = jnp.zeros((16,), jnp.float32)
            return 0

        lax.fori_loop(0, nkk, zS, 0)
        minf = jnp.full((16,), -jnp.inf, jnp.float32)
        zero16 = jnp.zeros((16,), jnp.float32)
        lanes = lax.iota(jnp.int32, 16)
        e_lo = segbuf[pl.ds(0, 16)][0]
        win0_i = jnp.minimum((e_lo // 8) * 8, WCLAMP)
        pltpu.sync_copy(srcs_hbm.at[pl.ds(win0_i, SBUFSZ)], srcbuf)

        def blk_body(blk, win0_b):
            b0 = d_lo + blk * 16
            pltpu.sync_copy(xr_hbm.at[pl.ds(b0, 16)], xrbuf)

            def dst_body(db, win0_d):
                rd = blk * 16 + db
                sv = segbuf[pl.ds(rd, 16)]
                e0 = sv[0]
                e1 = sv[1]
                ngr = (e1 - e0 + 15) // 16

                def group_body(g, car):
                    m, dsum, win0 = car
                    e = e0 + g * 16
                    need = (e - win0) + 16 > SBUFSZ
                    win0n = jnp.where(
                        need, jnp.minimum((e // 8) * 8, WCLAMP), win0)

                    @pl.when(need)
                    def _():
                        pltpu.sync_copy(
                            srcs_hbm.at[pl.ds(pl.multiple_of(win0n, 8),
                                              SBUFSZ)], srcbuf)

                    idxbuf[...] = srcbuf[pl.ds(e - win0n, 16)]
                    pltpu.async_copy(xl_hbm.at[idxbuf], rowbuf, sem).wait()
                    cnt = jnp.minimum(16, e1 - e)

                    def apass(kk, accs):
                        sl = pl.ds(kk * 16, 16)
                        attv = attb[sl]
                        xrv = xrbuf[db, sl]
                        out = []
                        for r in range(16):
                            z = rowbuf[r, sl] + xrv
                            l = 0.6 * z + 0.4 * jnp.abs(z)
                            out.append(accs[r] + attv * l)
                        return tuple(out)

                    accs = lax.fori_loop(0, nkk, apass, (zero16,) * 16)
                    a16 = minf
                    for r in range(16):
                        hr = jnp.full((16,), jnp.sum(accs[r]), jnp.float32)
                        a16 = jnp.where(lanes == r, hr, a16)
                    a16 = jnp.where(lanes < cnt, a16, minf)
                    gm = jnp.full((16,), jnp.max(a16), jnp.float32)
                    mn = jnp.maximum(m, gm)
                    rsc = jnp.exp(m - mn)
                    wv = jnp.exp(a16 - mn)
                    dsum = dsum * rsc + jnp.full(
                        (16,), jnp.sum(wv), jnp.float32)
                    ws = [jnp.full((16,), wv[r], jnp.float32)
                          for r in range(16)]

                    def spass(kk, _):
                        sl = pl.ds(kk * 16, 16)
                        sv2 = S[sl] * rsc
                        for r in range(16):
                            sv2 = sv2 + ws[r] * rowbuf[r, sl]
                        S[sl] = sv2
                        return 0

                    lax.fori_loop(0, nkk, spass, 0)
                    return (mn, dsum, win0n)

                m, dsum, win0_d = lax.fori_loop(
                    0, ngr, group_body, (minf, zero16, win0_d))
                rcp = jnp.where(dsum > 0, 1.0 / (dsum + 1e-16), 0.0)

                def flush(kk, _):
                    sl = pl.ds(kk * 16, 16)
                    outbuf[db, sl] = S[sl] * rcp + biasb[sl]
                    return 0

                lax.fori_loop(0, nkk, flush, 0)
                return win0_d

            win0_b = lax.fori_loop(0, 16, dst_body, win0_b)
            pltpu.sync_copy(outbuf, out_hbm.at[pl.ds(b0, 16)])
            return win0_b

        lax.fori_loop(0, DPW // 16, blk_body, win0_i)

    return k


def _pad_c(a, C):
    pad = C - a.shape[-1]
    if pad == 0:
        return a
    cfg = [(0, 0)] * (a.ndim - 1) + [(0, pad)]
    return jnp.pad(a, cfg)


def kernel(x, edge_index, W_src1, b_src1, W_dst1, b_dst1, att1, bias1,
           W_src2, b_src2, W_dst2, b_dst2, att2, bias2,
           W_src3, b_src3, W_dst3, b_dst3, att3, bias3):
    src = edge_index[0]
    dst = edge_index[1]
    srcs_s, seg = _sort_edges(src, dst)

    h = jnp.pad(x, ((0, N2 - N), (0, 0)))
    layers = [
        (W_src1, b_src1, W_dst1, b_dst1, att1, bias1, 128, False),
        (W_src2, b_src2, W_dst2, b_dst2, att2, bias2, 512, True),
        (W_src3, b_src3, W_dst3, b_dst3, att3, bias3, 1152, True),
    ]
    for (Ws, bs, Wd, bd, att, bias, C, leak) in layers:
        Ws, bs, Wd, bd = (_pad_c(Ws, C), _pad_c(bs, C),
                          _pad_c(Wd, C), _pad_c(bd, C))
        attp = _pad_c(att, C)[0]
        biasp = _pad_c(bias, C)
        xl, xr = _tc_linear(h, Ws, bs, Wd, bd, leak)
        h = _edge_kernel(C)(xl, xr, srcs_s, seg, attp, biasp)
    return h[:N, :1028]


# speedup vs baseline: 5.9740x; 1.1982x over previous
import functools

import jax
import jax.numpy as jnp
from jax import lax
from jax.experimental import pallas as pl
from jax.experimental.pallas import tpu as pltpu
from jax.experimental.pallas import tpu_sc as plsc

N = 10000
N2 = 10240
E = 160000
NW = 32
EPW = E // NW
CB = 2000
SEGSZ = 10368
DPW = 320

_mesh = lambda: plsc.VectorSubcoreMesh(core_axis_name="c", subcore_axis_name="s")


def _wid():
    return lax.axis_index("c") * 16 + lax.axis_index("s")


NGRP = (EPW + 15) // 16


@functools.partial(
    pl.kernel,
    out_type=jax.ShapeDtypeStruct((NW * N,), jnp.int32),
    mesh=_mesh(),
    compiler_params=pltpu.CompilerParams(needs_layout_passes=False),
    scratch_types=[pltpu.VMEM((EPW + 16,), jnp.int32),
                   pltpu.VMEM((N,), jnp.int32)],
)
def _hist_kernel(dst_hbm, hist_hbm, dstbuf, cnt):
    w = _wid()
    dstbuf[pl.ds(EPW, 16)] = jnp.zeros((16,), jnp.int32)
    pltpu.sync_copy(dst_hbm.at[pl.ds(w * EPW, EPW)], dstbuf.at[pl.ds(0, EPW)])

    def zbody(i, _):
        cnt[pl.ds(i * 16, 16)] = jnp.zeros((16,), jnp.int32)
        return 0

    lax.fori_loop(0, N // 16, zbody, 0)
    lanes = lax.iota(jnp.int32, 16)

    def ebody(j, _):
        idx = dstbuf[pl.ds(j * 16, 16)]
        valid = (j * 16 + lanes) < EPW
        g = plsc.load_gather(cnt, [idx])
        rc, last = plsc.scan_count(idx, mask=valid)
        plsc.store_scatter(cnt, [idx], g + rc, mask=last)
        return 0

    lax.fori_loop(0, NGRP, ebody, 0)
    pltpu.sync_copy(cnt, hist_hbm.at[pl.ds(w * N, N)])


@functools.partial(
    pl.kernel,
    out_type=(
        jax.ShapeDtypeStruct((NW * N,), jnp.int32),
        jax.ShapeDtypeStruct((SEGSZ,), jnp.int32),
    ),
    mesh=_mesh(),
    compiler_params=pltpu.CompilerParams(needs_layout_passes=False),
    scratch_types=[
        pltpu.VMEM((NW * CB,), jnp.int32),
        pltpu.VMEM((CB,), jnp.int32),
        pltpu.VMEM((CB,), jnp.int32),
        pltpu.VMEM((16,), jnp.int32),
    ],
)
def _scan_kernel(hist_hbm, base_hbm, seg_hbm, bpart, rowbuf, colsum, tmp16):
    w = _wid()

    @pl.when(w == 0)
    def _():
        def chunk_body(ch, running):
            def z(i, _):
                colsum[pl.ds(i * 16, 16)] = jnp.zeros((16,), jnp.int32)
                return 0

            lax.fori_loop(0, CB // 16, z, 0)
            for s in range(NW):
                pltpu.sync_copy(hist_hbm.at[pl.ds(s * N + ch * CB, CB)], rowbuf)

                def acc(i, _):
                    v = colsum[pl.ds(i * 16, 16)]
                    bpart[pl.ds(s * CB + i * 16, 16)] = v
                    colsum[pl.ds(i * 16, 16)] = v + rowbuf[pl.ds(i * 16, 16)]
                    return 0

                lax.fori_loop(0, CB // 16, acc, 0)

            def sbody(i, run):
                v = colsum[pl.ds(i * 16, 16)]
                cs = plsc.cumsum(v)
                colsum[pl.ds(i * 16, 16)] = cs - v + run
                return run + jnp.sum(v)

            run2 = lax.fori_loop(0, CB // 16, sbody, running)
            pltpu.sync_copy(colsum, seg_hbm.at[pl.ds(ch * CB, CB)])
            for s in range(NW):
                def fin(i, _):
                    bpart[pl.ds(s * CB + i * 16, 16)] = (
                        bpart[pl.ds(s * CB + i * 16, 16)]
                        + colsum[pl.ds(i * 16, 16)]
                    )
                    return 0

                lax.fori_loop(0, CB // 16, fin, 0)
                pltpu.sync_copy(bpart.at[pl.ds(s * CB, CB)],
                                base_hbm.at[pl.ds(s * N + ch * CB, CB)])
            return run2

        lax.fori_loop(0, N // CB, chunk_body, 0)
        tmp16[...] = jnp.full((16,), E, jnp.int32)

        def fill(i, _):
            pltpu.sync_copy(tmp16, seg_hbm.at[pl.ds(N + i * 16, 16)])
            return 0

        lax.fori_loop(0, (SEGSZ - N) // 16, fill, 0)


@functools.partial(
    pl.kernel,
    out_type=jax.ShapeDtypeStruct((E + 16,), jnp.int32),
    mesh=_mesh(),
    compiler_params=pltpu.CompilerParams(needs_layout_passes=False),
    scratch_types=[
        pltpu.VMEM((EPW + 16,), jnp.int32),
        pltpu.VMEM((EPW + 16,), jnp.int32),
        pltpu.VMEM((N,), jnp.int32),
        pltpu.VMEM((EPW + 16,), jnp.int32),
        pltpu.VMEM((16,), jnp.int32),
    ],
)
def _place_kernel(dst_hbm, src_hbm, base_hbm, srcs_hbm,
                  dstbuf, srcbuf, basebuf, posbuf, tmp16):
    w = _wid()
    dstbuf[pl.ds(EPW, 16)] = jnp.zeros((16,), jnp.int32)
    pltpu.sync_copy(dst_hbm.at[pl.ds(w * EPW, EPW)], dstbuf.at[pl.ds(0, EPW)])
    pltpu.sync_copy(src_hbm.at[pl.ds(w * EPW, EPW)], srcbuf.at[pl.ds(0, EPW)])
    pltpu.sync_copy(base_hbm.at[pl.ds(w * N, N)], basebuf)
    lanes = lax.iota(jnp.int32, 16)

    @pl.when(w == 0)
    def _():
        tmp16[...] = jnp.zeros((16,), jnp.int32)
        pltpu.sync_copy(tmp16, srcs_hbm.at[pl.ds(E, 16)])

    def ebody(j, _):
        idx = dstbuf[pl.ds(j * 16, 16)]
        valid = (j * 16 + lanes) < EPW
        g = plsc.load_gather(basebuf, [idx])
        rc, last = plsc.scan_count(idx, mask=valid)
        pos = jnp.where(valid, g + rc - 1, -1)
        posbuf[pl.ds(j * 16, 16)] = pos
        plsc.store_scatter(basebuf, [idx], g + rc, mask=last)
        return 0

    lax.fori_loop(0, NGRP, ebody, 0)

    def sbody(j, _):
        idx = posbuf[pl.ds(j * 16, 16)]
        pltpu.sync_copy(srcbuf.at[pl.ds(j * 16, 16)],
                        srcs_hbm.at[plsc.Indices(idx, ignored_value=-1)])
        return 0

    lax.fori_loop(0, NGRP, sbody, 0)


def _sort_edges(src, dst):
    hist = _hist_kernel(dst)
    base, seg = _scan_kernel(hist)
    srcs_s = _place_kernel(dst, src, base)
    return srcs_s, seg


def _tc_linear(h, W_src, b_src, W_dst, b_dst, leak):
    n, fin = h.shape
    C = W_src.shape[1]
    BM = 1024

    def body(h_ref, ws_ref, bs_ref, wd_ref, bd_ref, xl_ref, xr_ref):
        a = h_ref[...]
        if leak:
            a = jnp.where(a > 0, a, 0.01 * a)
        xl_ref[...] = (
            jnp.dot(a, ws_ref[...], preferred_element_type=jnp.float32)
            + bs_ref[...]
        )
        xr_ref[...] = (
            jnp.dot(a, wd_ref[...], preferred_element_type=jnp.float32)
            + bd_ref[...]
        )

    return pl.pallas_call(
        body,
        grid=(n // BM,),
        in_specs=[
            pl.BlockSpec((BM, fin), lambda i: (i, 0)),
            pl.BlockSpec((fin, C), lambda i: (0, 0)),
            pl.BlockSpec((1, C), lambda i: (0, 0)),
            pl.BlockSpec((fin, C), lambda i: (0, 0)),
            pl.BlockSpec((1, C), lambda i: (0, 0)),
        ],
        out_specs=[
            pl.BlockSpec((BM, C), lambda i: (i, 0)),
            pl.BlockSpec((BM, C), lambda i: (i, 0)),
        ],
        out_shape=[jax.ShapeDtypeStruct((n, C), jnp.float32)] * 2,
    )(h, W_src, b_src.reshape(1, C), W_dst, b_dst.reshape(1, C))


SBUFSZ = 4096
WCLAMP = E + 16 - SBUFSZ


@functools.cache
def _edge_kernel(C):
    nkk = C // 16

    @functools.partial(
        pl.kernel,
        out_type=jax.ShapeDtypeStruct((N2, C), jnp.float32),
        mesh=_mesh(),
        compiler_params=pltpu.CompilerParams(needs_layout_passes=False),
        scratch_types=[
            pltpu.VMEM((384,), jnp.int32),
            pltpu.VMEM((SBUFSZ,), jnp.int32),
            pltpu.VMEM((2, 16), jnp.int32),
            pltpu.VMEM((2, 16, C), jnp.float32),
            pltpu.VMEM((16, C), jnp.float32),
            pltpu.VMEM((16, C), jnp.float32),
            pltpu.VMEM((C,), jnp.float32),
            pltpu.VMEM((C,), jnp.float32),
            pltpu.VMEM((C,), jnp.float32),
            pltpu.SemaphoreType.DMA,
            pltpu.SemaphoreType.DMA,
        ],
    )
    def k(xl_hbm, xr_hbm, srcs_hbm, seg_hbm, att_hbm, bias_hbm, out_hbm,
          segbuf, srcbuf, idxbuf, rowbuf, xrbuf, outbuf, S, attb, biasb,
          sem, psem):
        w = _wid()
        d_lo = w * DPW
        pltpu.sync_copy(seg_hbm.at[pl.ds(d_lo, 384)], segbuf)
        pltpu.sync_copy(att_hbm, attb)
        pltpu.sync_copy(bias_hbm, biasb)

        def zS(i, _):
            S[pl.ds(i * 16, 16)] = jnp.zeros((16,), jnp.float32)
            return 0

        lax.fori_loop(0, nkk, zS, 0)
        minf = jnp.full((16,), -jnp.inf, jnp.float32)
        zero16 = jnp.zeros((16,), jnp.float32)
        lanes = lax.iota(jnp.int32, 16)
        e_lo = segbuf[pl.ds(0, 16)][0]
        win0_i = jnp.minimum((e_lo // 8) * 8, WCLAMP)
        pltpu.sync_copy(srcs_hbm.at[pl.ds(win0_i, SBUFSZ)], srcbuf)

        def blk_body(blk, win0_b):
            b0 = d_lo + blk * 16
            pltpu.sync_copy(xr_hbm.at[pl.ds(b0, 16)], xrbuf)

            def dst_body(db, car_d):
                win0_d, pf = car_d
                rd = blk * 16 + db
                p = rd % 2
                q = 1 - p
                sv = segbuf[pl.ds(rd, 16)]
                e0 = sv[0]
                e1 = sv[1]
                ngr = (e1 - e0 + 15) // 16

                @pl.when(pf > 0)
                def _():
                    pltpu.make_async_copy(
                        xl_hbm.at[idxbuf.at[p]], rowbuf.at[p], psem).wait()

                can_pf = jnp.logical_and(
                    (e1 - win0_d) + 16 <= SBUFSZ, rd + 1 < DPW)

                @pl.when(can_pf)
                def _():
                    idxbuf[q, ...] = srcbuf[pl.ds(e1 - win0_d, 16)]
                    pltpu.async_copy(
                        xl_hbm.at[idxbuf.at[q]], rowbuf.at[q], psem)

                pf_n = jnp.where(can_pf, 1, 0)

                def group_body(g, car):
                    m, dsum, win0 = car
                    e = e0 + g * 16
                    need = (e - win0) + 16 > SBUFSZ
                    win0n = jnp.where(
                        need, jnp.minimum((e // 8) * 8, WCLAMP), win0)

                    @pl.when(need)
                    def _():
                        pltpu.sync_copy(
                            srcs_hbm.at[pl.ds(pl.multiple_of(win0n, 8),
                                              SBUFSZ)], srcbuf)

                    @pl.when(jnp.logical_or(g > 0, pf == 0))
                    def _():
                        idxbuf[p, ...] = srcbuf[pl.ds(e - win0n, 16)]
                        pltpu.async_copy(
                            xl_hbm.at[idxbuf.at[p]], rowbuf.at[p], sem).wait()

                    cnt = jnp.minimum(16, e1 - e)

                    def apass(kk, accs):
                        sl = pl.ds(kk * 16, 16)
                        attv = attb[sl]
                        xrv = xrbuf[db, sl]
                        out = []
                        for r in range(16):
                            z = rowbuf[p, r, sl] + xrv
                            l = 0.6 * z + 0.4 * jnp.abs(z)
                            out.append(accs[r] + attv * l)
                        return tuple(out)

                    accs = lax.fori_loop(0, nkk, apass, (zero16,) * 16)
                    a16 = minf
                    for r in range(16):
                        hr = jnp.full((16,), jnp.sum(accs[r]), jnp.float32)
                        a16 = jnp.where(lanes == r, hr, a16)
                    a16 = jnp.where(lanes < cnt, a16, minf)
                    gm = jnp.full((16,), jnp.max(a16), jnp.float32)
                    mn = jnp.maximum(m, gm)
                    rsc = jnp.exp(m - mn)
                    wv = jnp.exp(a16 - mn)
                    dsum = dsum * rsc + jnp.full(
                        (16,), jnp.sum(wv), jnp.float32)
                    ws = [jnp.full((16,), wv[r], jnp.float32)
                          for r in range(16)]

                    def spass(kk, _):
                        sl = pl.ds(kk * 16, 16)
                        sv2 = S[sl] * rsc
                        for r in range(16):
                            sv2 = sv2 + ws[r] * rowbuf[p, r, sl]
                        S[sl] = sv2
                        return 0

                    lax.fori_loop(0, nkk, spass, 0)
                    return (mn, dsum, win0n)

                m, dsum, win0_d = lax.fori_loop(
                    0, ngr, group_body, (minf, zero16, win0_d))
                del m
                rcp = jnp.where(dsum > 0, 1.0 / (dsum + 1e-16), 0.0)

                def flush(kk, _):
                    sl = pl.ds(kk * 16, 16)
                    outbuf[db, sl] = S[sl] * rcp + biasb[sl]
                    return 0

                lax.fori_loop(0, nkk, flush, 0)
                return (win0_d, pf_n)

            win0_b = lax.fori_loop(0, 16, dst_body, win0_b)
            pltpu.sync_copy(outbuf, out_hbm.at[pl.ds(b0, 16)])
            return win0_b

        lax.fori_loop(0, DPW // 16, blk_body,
                      (win0_i, jnp.int32(0)))

    return k


def _pad_c(a, C):
    pad = C - a.shape[-1]
    if pad == 0:
        return a
    cfg = [(0, 0)] * (a.ndim - 1) + [(0, pad)]
    return jnp.pad(a, cfg)


def kernel(x, edge_index, W_src1, b_src1, W_dst1, b_dst1, att1, bias1,
           W_src2, b_src2, W_dst2, b_dst2, att2, bias2,
           W_src3, b_src3, W_dst3, b_dst3, att3, bias3):
    src = edge_index[0]
    dst = edge_index[1]
    srcs_s, seg = _sort_edges(src, dst)

    h = jnp.pad(x, ((0, N2 - N), (0, 0)))
    layers = [
        (W_src1, b_src1, W_dst1, b_dst1, att1, bias1, 128, False),
        (W_src2, b_src2, W_dst2, b_dst2, att2, bias2, 512, True),
        (W_src3, b_src3, W_dst3, b_dst3, att3, bias3, 1152, True),
    ]
    for (Ws, bs, Wd, bd, att, bias, C, leak) in layers:
        Ws, bs, Wd, bd = (_pad_c(Ws, C), _pad_c(bs, C),
                          _pad_c(Wd, C), _pad_c(bd, C))
        attp = _pad_c(att, C)[0]
        biasp = _pad_c(bias, C)
        xl, xr = _tc_linear(h, Ws, bs, Wd, bd, leak)
        h = _edge_kernel(C)(xl, xr, srcs_s, seg, attp, biasp)
    return h[:N, :1028]


# speedup vs baseline: 6.0219x; 1.0080x over previous
import functools

import jax
import jax.numpy as jnp
from jax import lax
from jax.experimental import pallas as pl
from jax.experimental.pallas import tpu as pltpu
from jax.experimental.pallas import tpu_sc as plsc

N = 10000
N2 = 10240
E = 160000
NW = 32
EPW = E // NW
CB = 2000
SEGSZ = 10368
DPW = 320

_mesh = lambda: plsc.VectorSubcoreMesh(core_axis_name="c", subcore_axis_name="s")


def _wid():
    return lax.axis_index("c") * 16 + lax.axis_index("s")


NGRP = (EPW + 15) // 16


@functools.partial(
    pl.kernel,
    out_type=jax.ShapeDtypeStruct((NW * N,), jnp.int32),
    mesh=_mesh(),
    compiler_params=pltpu.CompilerParams(needs_layout_passes=False),
    scratch_types=[pltpu.VMEM((EPW + 16,), jnp.int32),
                   pltpu.VMEM((N,), jnp.int32)],
)
def _hist_kernel(dst_hbm, hist_hbm, dstbuf, cnt):
    w = _wid()
    dstbuf[pl.ds(EPW, 16)] = jnp.zeros((16,), jnp.int32)
    pltpu.sync_copy(dst_hbm.at[pl.ds(w * EPW, EPW)], dstbuf.at[pl.ds(0, EPW)])

    def zbody(i, _):
        cnt[pl.ds(i * 16, 16)] = jnp.zeros((16,), jnp.int32)
        return 0

    lax.fori_loop(0, N // 16, zbody, 0)
    lanes = lax.iota(jnp.int32, 16)

    def ebody(j, _):
        idx = dstbuf[pl.ds(j * 16, 16)]
        valid = (j * 16 + lanes) < EPW
        g = plsc.load_gather(cnt, [idx])
        rc, last = plsc.scan_count(idx, mask=valid)
        plsc.store_scatter(cnt, [idx], g + rc, mask=last)
        return 0

    lax.fori_loop(0, NGRP, ebody, 0)
    pltpu.sync_copy(cnt, hist_hbm.at[pl.ds(w * N, N)])


@functools.partial(
    pl.kernel,
    out_type=(
        jax.ShapeDtypeStruct((NW * N,), jnp.int32),
        jax.ShapeDtypeStruct((SEGSZ,), jnp.int32),
    ),
    mesh=_mesh(),
    compiler_params=pltpu.CompilerParams(needs_layout_passes=False),
    scratch_types=[
        pltpu.VMEM((NW * CB,), jnp.int32),
        pltpu.VMEM((CB,), jnp.int32),
        pltpu.VMEM((CB,), jnp.int32),
        pltpu.VMEM((16,), jnp.int32),
    ],
)
def _scan_kernel(hist_hbm, base_hbm, seg_hbm, bpart, rowbuf, colsum, tmp16):
    w = _wid()

    @pl.when(w == 0)
    def _():
        def chunk_body(ch, running):
            def z(i, _):
                colsum[pl.ds(i * 16, 16)] = jnp.zeros((16,), jnp.int32)
                return 0

            lax.fori_loop(0, CB // 16, z, 0)
            for s in range(NW):
                pltpu.sync_copy(hist_hbm.at[pl.ds(s * N + ch * CB, CB)], rowbuf)

                def acc(i, _):
                    v = colsum[pl.ds(i * 16, 16)]
                    bpart[pl.ds(s * CB + i * 16, 16)] = v
                    colsum[pl.ds(i * 16, 16)] = v + rowbuf[pl.ds(i * 16, 16)]
                    return 0

                lax.fori_loop(0, CB // 16, acc, 0)

            def sbody(i, run):
                v = colsum[pl.ds(i * 16, 16)]
                cs = plsc.cumsum(v)
                colsum[pl.ds(i * 16, 16)] = cs - v + run
                return run + jnp.sum(v)

            run2 = lax.fori_loop(0, CB // 16, sbody, running)
            pltpu.sync_copy(colsum, seg_hbm.at[pl.ds(ch * CB, CB)])
            for s in range(NW):
                def fin(i, _):
                    bpart[pl.ds(s * CB + i * 16, 16)] = (
                        bpart[pl.ds(s * CB + i * 16, 16)]
                        + colsum[pl.ds(i * 16, 16)]
                    )
                    return 0

                lax.fori_loop(0, CB // 16, fin, 0)
                pltpu.sync_copy(bpart.at[pl.ds(s * CB, CB)],
                                base_hbm.at[pl.ds(s * N + ch * CB, CB)])
            return run2

        lax.fori_loop(0, N // CB, chunk_body, 0)
        tmp16[...] = jnp.full((16,), E, jnp.int32)

        def fill(i, _):
            pltpu.sync_copy(tmp16, seg_hbm.at[pl.ds(N + i * 16, 16)])
            return 0

        lax.fori_loop(0, (SEGSZ - N) // 16, fill, 0)


@functools.partial(
    pl.kernel,
    out_type=jax.ShapeDtypeStruct((E + 16,), jnp.int32),
    mesh=_mesh(),
    compiler_params=pltpu.CompilerParams(needs_layout_passes=False),
    scratch_types=[
        pltpu.VMEM((EPW + 16,), jnp.int32),
        pltpu.VMEM((EPW + 16,), jnp.int32),
        pltpu.VMEM((N,), jnp.int32),
        pltpu.VMEM((EPW + 16,), jnp.int32),
        pltpu.VMEM((16,), jnp.int32),
    ],
)
def _place_kernel(dst_hbm, src_hbm, base_hbm, srcs_hbm,
                  dstbuf, srcbuf, basebuf, posbuf, tmp16):
    w = _wid()
    dstbuf[pl.ds(EPW, 16)] = jnp.zeros((16,), jnp.int32)
    pltpu.sync_copy(dst_hbm.at[pl.ds(w * EPW, EPW)], dstbuf.at[pl.ds(0, EPW)])
    pltpu.sync_copy(src_hbm.at[pl.ds(w * EPW, EPW)], srcbuf.at[pl.ds(0, EPW)])
    pltpu.sync_copy(base_hbm.at[pl.ds(w * N, N)], basebuf)
    lanes = lax.iota(jnp.int32, 16)

    @pl.when(w == 0)
    def _():
        tmp16[...] = jnp.zeros((16,), jnp.int32)
        pltpu.sync_copy(tmp16, srcs_hbm.at[pl.ds(E, 16)])

    def ebody(j, _):
        idx = dstbuf[pl.ds(j * 16, 16)]
        valid = (j * 16 + lanes) < EPW
        g = plsc.load_gather(basebuf, [idx])
        rc, last = plsc.scan_count(idx, mask=valid)
        pos = jnp.where(valid, g + rc - 1, -1)
        posbuf[pl.ds(j * 16, 16)] = pos
        plsc.store_scatter(basebuf, [idx], g + rc, mask=last)
        return 0

    lax.fori_loop(0, NGRP, ebody, 0)

    def sbody(j, _):
        idx = posbuf[pl.ds(j * 16, 16)]
        pltpu.sync_copy(srcbuf.at[pl.ds(j * 16, 16)],
                        srcs_hbm.at[plsc.Indices(idx, ignored_value=-1)])
        return 0

    lax.fori_loop(0, NGRP, sbody, 0)


def _sort_edges(src, dst):
    hist = _hist_kernel(dst)
    base, seg = _scan_kernel(hist)
    srcs_s = _place_kernel(dst, src, base)
    return srcs_s, seg


def _tc_linear(h, W_src, b_src, W_dst, b_dst, leak):
    n, fin = h.shape
    C = W_src.shape[1]
    BM = 1024

    def body(h_ref, ws_ref, bs_ref, wd_ref, bd_ref, xl_ref, xr_ref):
        a = h_ref[...]
        if leak:
            a = jnp.where(a > 0, a, 0.01 * a)
        xl_ref[...] = (
            jnp.dot(a, ws_ref[...], preferred_element_type=jnp.float32)
            + bs_ref[...]
        )
        xr_ref[...] = (
            jnp.dot(a, wd_ref[...], preferred_element_type=jnp.float32)
            + bd_ref[...]
        )

    return pl.pallas_call(
        body,
        grid=(n // BM,),
        in_specs=[
            pl.BlockSpec((BM, fin), lambda i: (i, 0)),
            pl.BlockSpec((fin, C), lambda i: (0, 0)),
            pl.BlockSpec((1, C), lambda i: (0, 0)),
            pl.BlockSpec((fin, C), lambda i: (0, 0)),
            pl.BlockSpec((1, C), lambda i: (0, 0)),
        ],
        out_specs=[
            pl.BlockSpec((BM, C), lambda i: (i, 0)),
            pl.BlockSpec((BM, C), lambda i: (i, 0)),
        ],
        out_shape=[jax.ShapeDtypeStruct((n, C), jnp.float32)] * 2,
    )(h, W_src, b_src.reshape(1, C), W_dst, b_dst.reshape(1, C))


SBUFSZ = 4096
WCLAMP = E + 16 - SBUFSZ


@functools.cache
def _edge_kernel(C):
    nkk = C // 16

    @functools.partial(
        pl.kernel,
        out_type=jax.ShapeDtypeStruct((N2, C), jnp.float32),
        mesh=_mesh(),
        compiler_params=pltpu.CompilerParams(needs_layout_passes=False),
        scratch_types=[
            pltpu.VMEM((384,), jnp.int32),
            pltpu.VMEM((SBUFSZ,), jnp.int32),
            pltpu.VMEM((2, 16), jnp.int32),
            pltpu.VMEM((2, 16, C), jnp.float32),
            pltpu.VMEM((16, C), jnp.float32),
            pltpu.VMEM((16, C), jnp.float32),
            pltpu.VMEM((C,), jnp.float32),
            pltpu.VMEM((C,), jnp.float32),
            pltpu.VMEM((C,), jnp.float32),
            pltpu.SemaphoreType.DMA,
            pltpu.SemaphoreType.DMA,
        ],
    )
    def k(xl_hbm, xr_hbm, srcs_hbm, seg_hbm, att_hbm, bias_hbm, out_hbm,
          segbuf, srcbuf, idxbuf, rowbuf, xrbuf, outbuf, S, attb, biasb,
          sem, psem):
        w = _wid()
        d_lo = w * DPW
        pltpu.sync_copy(seg_hbm.at[pl.ds(d_lo, 384)], segbuf)
        pltpu.sync_copy(att_hbm, attb)
        pltpu.sync_copy(bias_hbm, biasb)

        def zS(i, _):
            S[pl.ds(i * 16, 16)] = jnp.zeros((16,), jnp.float32)
            return 0

        lax.fori_loop(0, nkk, zS, 0)
        minf = jnp.full((16,), -jnp.inf, jnp.float32)
        zero16 = jnp.zeros((16,), jnp.float32)
        lanes = lax.iota(jnp.int32, 16)
        e_lo = segbuf[pl.ds(0, 16)][0]
        win0_i = jnp.minimum((e_lo // 8) * 8, WCLAMP)
        pltpu.sync_copy(srcs_hbm.at[pl.ds(win0_i, SBUFSZ)], srcbuf)

        def blk_body(blk, win0_b):
            b0 = d_lo + blk * 16
            pltpu.sync_copy(xr_hbm.at[pl.ds(b0, 16)], xrbuf)

            def dst_body(db, car_d):
                win0_d, pf = car_d
                rd = blk * 16 + db
                p = rd % 2
                q = 1 - p
                sv = segbuf[pl.ds(rd, 16)]
                e0 = sv[0]
                e1 = sv[1]
                ngr = (e1 - e0 + 15) // 16

                @pl.when(pf > 0)
                def _():
                    pltpu.make_async_copy(
                        xl_hbm.at[idxbuf.at[p]], rowbuf.at[p], psem).wait()

                can_pf = jnp.logical_and(
                    (e1 - win0_d) + 16 <= SBUFSZ, rd + 1 < DPW)

                @pl.when(can_pf)
                def _():
                    idxbuf[q, ...] = srcbuf[pl.ds(e1 - win0_d, 16)]
                    pltpu.async_copy(
                        xl_hbm.at[idxbuf.at[q]], rowbuf.at[q], psem)

                pf_n = jnp.where(can_pf, 1, 0)

                def group_body(g, car):
                    m, dsum, win0 = car
                    e = e0 + g * 16
                    need = (e - win0) + 16 > SBUFSZ
                    win0n = jnp.where(
                        need, jnp.minimum((e // 8) * 8, WCLAMP), win0)

                    @pl.when(need)
                    def _():
                        pltpu.sync_copy(
                            srcs_hbm.at[pl.ds(pl.multiple_of(win0n, 8),
                                              SBUFSZ)], srcbuf)

                    @pl.when(jnp.logical_or(g > 0, pf == 0))
                    def _():
                        idxbuf[p, ...] = srcbuf[pl.ds(e - win0n, 16)]
                        pltpu.async_copy(
                            xl_hbm.at[idxbuf.at[p]], rowbuf.at[p], sem).wait()

                    cnt = jnp.minimum(16, e1 - e)

                    def apass(kk, accs):
                        sl = pl.ds(kk * 16, 16)
                        attv = attb[sl]
                        xrv = xrbuf[db, sl]
                        out = []
                        for r in range(16):
                            z = rowbuf[p, r, sl] + xrv
                            l = 0.6 * z + 0.4 * jnp.abs(z)
                            out.append(accs[r] + attv * l)
                        return tuple(out)

                    accs = lax.fori_loop(0, nkk, apass, (zero16,) * 16)
                    a16 = minf
                    for r in range(16):
                        hr = jnp.full((16,), jnp.sum(accs[r]), jnp.float32)
                        a16 = jnp.where(lanes == r, hr, a16)
                    a16 = jnp.where(lanes < cnt, a16, minf)
                    gm = jnp.full((16,), jnp.max(a16), jnp.float32)
                    mn = jnp.maximum(m, gm)
                    rsc = jnp.exp(m - mn)
                    wv = jnp.exp(a16 - mn)
                    dsum = dsum * rsc + jnp.full(
                        (16,), jnp.sum(wv), jnp.float32)
                    ws = [jnp.full((16,), wv[r], jnp.float32)
                          for r in range(16)]

                    @pl.when(g == 0)
                    def _():
                        def spass0(kk, _):
                            sl = pl.ds(kk * 16, 16)
                            sv2 = ws[0] * rowbuf[p, 0, sl]
                            for r in range(1, 16):
                                sv2 = sv2 + ws[r] * rowbuf[p, r, sl]
                            S[sl] = sv2
                            return 0

                        lax.fori_loop(0, nkk, spass0, 0)

                    @pl.when(g > 0)
                    def _():
                        def spass(kk, _):
                            sl = pl.ds(kk * 16, 16)
                            sv2 = S[sl] * rsc
                            for r in range(16):
                                sv2 = sv2 + ws[r] * rowbuf[p, r, sl]
                            S[sl] = sv2
                            return 0

                        lax.fori_loop(0, nkk, spass, 0)
                    return (mn, dsum, win0n)

                m, dsum, win0_d = lax.fori_loop(
                    0, ngr, group_body, (minf, zero16, win0_d))
                del m
                rcp = jnp.where(dsum > 0, 1.0 / (dsum + 1e-16), 0.0)

                def flush(kk, _):
                    sl = pl.ds(kk * 16, 16)
                    outbuf[db, sl] = S[sl] * rcp + biasb[sl]
                    return 0

                lax.fori_loop(0, nkk, flush, 0)
                return (win0_d, pf_n)

            win0_b = lax.fori_loop(0, 16, dst_body, win0_b)
            pltpu.sync_copy(outbuf, out_hbm.at[pl.ds(b0, 16)])
            return win0_b

        lax.fori_loop(0, DPW // 16, blk_body,
                      (win0_i, jnp.int32(0)))

    return k


def _pad_c(a, C):
    pad = C - a.shape[-1]
    if pad == 0:
        return a
    cfg = [(0, 0)] * (a.ndim - 1) + [(0, pad)]
    return jnp.pad(a, cfg)


def kernel(x, edge_index, W_src1, b_src1, W_dst1, b_dst1, att1, bias1,
           W_src2, b_src2, W_dst2, b_dst2, att2, bias2,
           W_src3, b_src3, W_dst3, b_dst3, att3, bias3):
    src = edge_index[0]
    dst = edge_index[1]
    srcs_s, seg = _sort_edges(src, dst)

    h = jnp.pad(x, ((0, N2 - N), (0, 0)))
    layers = [
        (W_src1, b_src1, W_dst1, b_dst1, att1, bias1, 128, False),
        (W_src2, b_src2, W_dst2, b_dst2, att2, bias2, 512, True),
        (W_src3, b_src3, W_dst3, b_dst3, att3, bias3, 1152, True),
    ]
    for (Ws, bs, Wd, bd, att, bias, C, leak) in layers:
        Ws, bs, Wd, bd = (_pad_c(Ws, C), _pad_c(bs, C),
                          _pad_c(Wd, C), _pad_c(bd, C))
        attp = _pad_c(att, C)[0]
        biasp = _pad_c(bias, C)
        xl, xr = _tc_linear(h, Ws, bs, Wd, bd, leak)
        h = _edge_kernel(C)(xl, xr, srcs_s, seg, attp, biasp)
    return h[:N, :1028]
